# baseline (speedup 1.0000x reference)
import functools

import jax
import jax.numpy as jnp
from jax import lax
from jax.experimental import pallas as pl
from jax.experimental.pallas import tpu as pltpu
from jax.experimental.pallas import tpu_sc as plsc

B, T, E = 1024, 200, 128
NC, NS = 2, 16
NW = NC * NS
ROWS_PW = B // NW
N = B * T
IA, IB = 128, T - 128
NBUF = 4
GROUPS = ROWS_PW // NBUF

_mesh = plsc.VectorSubcoreMesh(core_axis_name="c", subcore_axis_name="s")


@functools.partial(
    pl.kernel,
    out_type=jax.ShapeDtypeStruct((N, E), jnp.float32),
    mesh=_mesh,
    scratch_types=(
        [pltpu.VMEM_SHARED((T, E), jnp.float32)]
        + [pltpu.VMEM((T, E), jnp.float32)] * NBUF
        + [pltpu.VMEM((ROWS_PW * T,), jnp.int32)]
        + [pltpu.SemaphoreType.DMA] * (3 * NBUF)
    ),
)
def _emb_kernel(x_hbm, tok_hbm, pos_hbm, out_hbm, pos_sh, *scratch):
    rows = scratch[:NBUF]
    idx_v = scratch[NBUF]
    sem_g = scratch[NBUF + 1:2 * NBUF + 1]
    sem_w = scratch[2 * NBUF + 1:3 * NBUF + 1]
    sem_r = scratch[3 * NBUF + 1:]

    sid = lax.axis_index("s")
    wid = sid * NC + lax.axis_index("c")

    @pl.when(sid == 0)
    def _stage_pos():
        pltpu.sync_copy(pos_hbm.at[pl.ds(0, T)], pos_sh)

    plsc.subcore_barrier()

    cbase = wid * ROWS_PW

    pltpu.sync_copy(x_hbm.at[pl.ds(cbase * T, ROWS_PW * T)], idx_v)

    def refill_and_gather(p, off):
        pltpu.make_async_copy(pos_sh, rows[p], sem_r[p]).wait()
        return (
            pltpu.async_copy(tok_hbm.at[idx_v.at[pl.ds(off, IA)]],
                             rows[p].at[pl.ds(0, IA)], sem_g[p], add=True),
            pltpu.async_copy(tok_hbm.at[idx_v.at[pl.ds(off + IA, IB)]],
                             rows[p].at[pl.ds(IA, IB)], sem_g[p], add=True),
        )

    def start_writes(base, gs):
        for p in range(NBUF):
            for c in gs[p]:
                c.wait()
            pltpu.async_copy(rows[p], out_hbm.at[pl.ds(base + p * T, T)],
                             sem_w[p])

    for p in range(NBUF):
        pltpu.async_copy(pos_sh, rows[p], sem_r[p])
    start_writes(cbase * T,
                 [refill_and_gather(p, p * T) for p in range(NBUF)])

    def body(g, _):
        base = (cbase + NBUF * g) * T
        off = NBUF * g * T
        for p in range(NBUF):
            pltpu.make_async_copy(
                rows[p], out_hbm.at[pl.ds(base + (p - NBUF) * T, T)],
                sem_w[p]).wait()
            pltpu.async_copy(pos_sh, rows[p], sem_r[p])
        start_writes(base, [refill_and_gather(p, off + p * T)
                            for p in range(NBUF)])
        return 0

    lax.fori_loop(1, GROUPS, body, 0)

    last = (cbase + ROWS_PW - NBUF) * T
    for p in range(NBUF):
        pltpu.make_async_copy(rows[p], out_hbm.at[pl.ds(last + p * T, T)],
                              sem_w[p]).wait()


def kernel(x, tok_table, pos_table):
    out = _emb_kernel(x.reshape(-1).astype(jnp.int32), tok_table, pos_table)
    return out.reshape(B, T, E)


# speedup vs baseline: 1.0464x; 1.0464x over previous
import functools

import jax
import jax.numpy as jnp
from jax import lax
from jax.experimental import pallas as pl
from jax.experimental.pallas import tpu as pltpu
from jax.experimental.pallas import tpu_sc as plsc

B, T, E = 1024, 200, 128
NC, NS = 2, 16
NW = NC * NS
ROWS_PW = B // NW
N = B * T
IA, IB = 104, T - 104
NBUF = 4
GROUPS = ROWS_PW // NBUF

_mesh = plsc.VectorSubcoreMesh(core_axis_name="c", subcore_axis_name="s")


@functools.partial(
    pl.kernel,
    out_type=jax.ShapeDtypeStruct((N, E), jnp.float32),
    mesh=_mesh,
    scratch_types=[
        pltpu.VMEM_SHARED((T, E), jnp.float32),
        pltpu.VMEM((T, E), jnp.float32),
        pltpu.VMEM((T, E), jnp.float32),
        pltpu.VMEM((T, E), jnp.float32),
        pltpu.VMEM((T, E), jnp.float32),
        pltpu.VMEM((ROWS_PW * T,), jnp.int32),
        pltpu.SemaphoreType.DMA,
        pltpu.SemaphoreType.DMA,
        pltpu.SemaphoreType.DMA,
        pltpu.SemaphoreType.DMA,
        pltpu.SemaphoreType.DMA,
        pltpu.SemaphoreType.DMA,
        pltpu.SemaphoreType.DMA,
        pltpu.SemaphoreType.DMA,
    ],
)
def _emb_kernel(x_hbm, tok_hbm, pos_hbm, out_hbm,
                pos_sh, rows0, rows1, rows2, rows3, idx_v,
                sem_g0, sem_g1, sem_g2, sem_g3,
                sem_w0, sem_w1, sem_w2, sem_w3):
    rows = (rows0, rows1, rows2, rows3)
    sem_g = (sem_g0, sem_g1, sem_g2, sem_g3)
    sem_w = (sem_w0, sem_w1, sem_w2, sem_w3)

    sid = lax.axis_index("s")
    wid = sid * NC + lax.axis_index("c")

    @pl.when(sid == 0)
    def _stage_pos():
        pltpu.sync_copy(pos_hbm.at[pl.ds(0, T)], pos_sh)

    plsc.subcore_barrier()

    cbase = wid * ROWS_PW

    pltpu.sync_copy(x_hbm.at[pl.ds(cbase * T, ROWS_PW * T)], idx_v)

    def refill_and_gather(p, off):
        pltpu.sync_copy(pos_sh, rows[p])
        return (
            pltpu.async_copy(tok_hbm.at[idx_v.at[pl.ds(off, IA)]],
                             rows[p].at[pl.ds(0, IA)], sem_g[p], add=True),
            pltpu.async_copy(tok_hbm.at[idx_v.at[pl.ds(off + IA, IB)]],
                             rows[p].at[pl.ds(IA, IB)], sem_g[p], add=True),
        )

    def start_writes(base, gs):
        for p in range(NBUF):
            for c in gs[p]:
                c.wait()
            pltpu.async_copy(rows[p], out_hbm.at[pl.ds(base + p * T, T)],
                             sem_w[p])

    start_writes(cbase * T,
                 [refill_and_gather(p, p * T) for p in range(NBUF)])

    def body(g, _):
        base = (cbase + NBUF * g) * T
        off = NBUF * g * T
        gs = []
        for p in range(NBUF):
            pltpu.make_async_copy(
                rows[p], out_hbm.at[pl.ds(base + (p - NBUF) * T, T)],
                sem_w[p]).wait()
            gs.append(refill_and_gather(p, off + p * T))
        start_writes(base, gs)
        return 0

    lax.fori_loop(1, GROUPS, body, 0)

    last = (cbase + ROWS_PW - NBUF) * T
    for p in range(NBUF):
        pltpu.make_async_copy(rows[p], out_hbm.at[pl.ds(last + p * T, T)],
                              sem_w[p]).wait()


def kernel(x, tok_table, pos_table):
    out = _emb_kernel(x.reshape(-1).astype(jnp.int32), tok_table, pos_table)
    return out.reshape(B, T, E)
